# baseline (speedup 1.0000x reference)
import functools

import jax
import jax.numpy as jnp
from jax import lax
from jax.experimental import pallas as pl
from jax.experimental.pallas import tpu as pltpu
from jax.experimental.pallas import tpu_sc as plsc

_NBUF = 3
_CH = 16


@functools.lru_cache(maxsize=None)
def _make_gather(BT: int, S: int, D: int):
    info = plsc.get_sparse_core_info()
    NC, NS = info.num_cores, info.num_subcores
    NW = NC * NS
    B = BT * S
    b_per_w = B // NW
    w_per_row = S // b_per_w
    n_chunks = b_per_w // _CH
    assert n_chunks % _NBUF == 1 and n_chunks >= 2 * _NBUF
    n_full_rings = (n_chunks - 1) // _NBUF
    mesh = plsc.VectorSubcoreMesh(core_axis_name="c", subcore_axis_name="s")

    @functools.partial(
        pl.kernel,
        mesh=mesh,
        out_type=jax.ShapeDtypeStruct((B, D), jnp.float32),
        scratch_types=[
            pltpu.VMEM((b_per_w,), jnp.int32),
            *[pltpu.VMEM((_CH, D), jnp.float32) for _ in range(_NBUF)],
            *[pltpu.SemaphoreType.DMA for _ in range(2 * _NBUF)],
        ],
    )
    def gather_kernel(idx_hbm, table_hbm, out_hbm, idx_v, *bufs_and_sems):
        bufs = bufs_and_sems[:_NBUF]
        gsem = bufs_and_sems[_NBUF : 2 * _NBUF]
        osem = bufs_and_sems[2 * _NBUF :]
        wid = lax.axis_index("s") * NC + lax.axis_index("c")
        base = wid * b_per_w
        pltpu.sync_copy(
            idx_hbm.at[wid // w_per_row, pl.ds((wid % w_per_row) * b_per_w, b_per_w)],
            idx_v,
        )

        def start_gather(c, b):
            pltpu.async_copy(
                table_hbm.at[idx_v.at[pl.ds(c * _CH, _CH)]], bufs[b], gsem[b]
            )

        def wait_gather(c, b):
            pltpu.make_async_copy(
                table_hbm.at[idx_v.at[pl.ds(c * _CH, _CH)]], bufs[b], gsem[b]
            ).wait()

        def start_out(c, b):
            pltpu.async_copy(
                bufs[b], out_hbm.at[pl.ds(base + c * _CH, _CH)], osem[b]
            )

        def wait_out(c, b):
            pltpu.make_async_copy(
                bufs[b], out_hbm.at[pl.ds(base + c * _CH, _CH)], osem[b]
            ).wait()

        start_gather(0, 0)
        start_gather(1, 1)

        start_gather(2, 2)
        wait_gather(0, 0)
        start_out(0, 0)
        for b in (1, 2):
            c = b
            wait_out(c - 1, (b + 2) % _NBUF)
            start_gather(c + 2, (b + 2) % _NBUF)
            wait_gather(c, b)
            start_out(c, b)

        def ring(r, carry):
            for b in range(_NBUF):
                c = r * _NBUF + b
                wait_out(c - 1, (b + 2) % _NBUF)
                start_gather(c + 2, (b + 2) % _NBUF)
                wait_gather(c, b)
                start_out(c, b)
            return carry

        lax.fori_loop(1, n_full_rings - 1, ring, 0)

        for c in range((n_full_rings - 1) * _NBUF, n_chunks):
            b = c % _NBUF
            if c + 2 < n_chunks:
                wait_out(c - 1, (b + 2) % _NBUF)
                start_gather(c + 2, (b + 2) % _NBUF)
            wait_gather(c, b)
            start_out(c, b)

        for c in range(n_chunks - _NBUF, n_chunks):
            wait_out(c, c % _NBUF)

    return gather_kernel


def kernel(input_ids, wte):
    BT, S = input_ids.shape
    D = wte.shape[1]
    if input_ids.dtype != jnp.int32:
        input_ids = input_ids.astype(jnp.int32)
    out = _make_gather(BT, S, D)(input_ids, wte)
    return out.reshape(BT, S, D)
